# baseline (speedup 1.0000x reference)
import functools

import jax
import jax.numpy as jnp
from jax.experimental import pallas as pl
from jax.experimental.pallas import tpu as pltpu

E = 16
TOPK = 2
D = 1024
F = 4096
T = 256

TR = 32
CAP = T * TOPK + E * TR
FT = 512
NF = F // FT


def _ffn_body(astart_ref, nt_ref, xs_ref, w1_ref, w3_ref, w2_ref, ys_ref):
    e = pl.program_id(0)
    f = pl.program_id(1)
    base = astart_ref[e]
    nt = nt_ref[e]
    w1b = w1_ref[0].astype(jnp.bfloat16)
    w3b = w3_ref[0].astype(jnp.bfloat16)
    w2b = w2_ref[0].astype(jnp.bfloat16)

    def tile(i, carry):
        r0 = pl.multiple_of(base + i * TR, TR)
        xt = xs_ref[pl.ds(r0, TR), :].astype(jnp.bfloat16)
        h = jnp.dot(xt, w1b, preferred_element_type=jnp.float32)
        g = jnp.dot(xt, w3b, preferred_element_type=jnp.float32)
        act = (h * jax.nn.sigmoid(h)) * g
        y = jnp.dot(act.astype(jnp.bfloat16), w2b,
                    preferred_element_type=jnp.float32)

        @pl.when(f == 0)
        def _():
            ys_ref[pl.ds(r0, TR), :] = y

        @pl.when(f != 0)
        def _():
            ys_ref[pl.ds(r0, TR), :] = ys_ref[pl.ds(r0, TR), :] + y

        return carry

    jax.lax.fori_loop(0, nt, tile, 0)


def _expert_ffn(xs, w1, w3, w2, astart, ntiles):
    return pl.pallas_call(
        _ffn_body,
        grid=(E, NF),
        in_specs=[
            pl.BlockSpec(memory_space=pltpu.SMEM),
            pl.BlockSpec(memory_space=pltpu.SMEM),
            pl.BlockSpec((CAP, D), lambda e, f: (0, 0)),
            pl.BlockSpec((1, D, FT), lambda e, f: (e, 0, f)),
            pl.BlockSpec((1, D, FT), lambda e, f: (e, 0, f)),
            pl.BlockSpec((1, FT, D), lambda e, f: (e, f, 0)),
        ],
        out_specs=pl.BlockSpec((CAP, D), lambda e, f: (0, 0)),
        out_shape=jax.ShapeDtypeStruct((CAP, D), jnp.float32),
    )(astart, ntiles, xs, w1, w3, w2)


def kernel(hidden_states, gate_w, w1, w3, w2):
    b, s, d = hidden_states.shape
    x = hidden_states.reshape(-1, d)

    logits = x @ gate_w
    m0 = jnp.max(logits, axis=1)
    e0 = jnp.argmax(logits, axis=1).astype(jnp.int32)
    masked = jnp.where(jax.nn.one_hot(e0, E, dtype=jnp.bool_), -jnp.inf, logits)
    m1 = jnp.max(masked, axis=1)
    e1 = jnp.argmax(masked, axis=1).astype(jnp.int32)
    wn0 = 1.0 / (1.0 + jnp.exp(m1 - m0))
    wn1 = 1.0 - wn0

    eid = jnp.concatenate([e0, e1])
    onehot = jax.nn.one_hot(eid, E, dtype=jnp.int32)
    counts = onehot.sum(axis=0)
    aligned = ((counts + TR - 1) // TR) * TR
    astart = jnp.concatenate([jnp.zeros((1,), jnp.int32),
                              jnp.cumsum(aligned)[:-1].astype(jnp.int32)])
    ntiles = (aligned // TR).astype(jnp.int32)
    rank = (jnp.cumsum(onehot, axis=0) - onehot)
    rank = jnp.take_along_axis(rank, eid[:, None], axis=1)[:, 0]
    pos = astart[eid] + rank.astype(jnp.int32)
    tok = jnp.mod(jnp.arange(2 * T, dtype=jnp.int32), T)
    tok_ids = jnp.zeros((CAP,), jnp.int32).at[pos].set(tok)
    slot0, slot1 = pos[:T], pos[T:]

    xs = x[tok_ids]

    ys = _expert_ffn(xs, w1, w3, w2, astart, ntiles)

    out = wn0[:, None] * ys[slot0] + wn1[:, None] * ys[slot1]
    return out.reshape(b, s, d).astype(hidden_states.dtype), logits


# speedup vs baseline: 1.2110x; 1.2110x over previous
import functools

import jax
import jax.numpy as jnp
from jax.experimental import pallas as pl
from jax.experimental.pallas import tpu as pltpu

E = 16
TOPK = 2
D = 1024
F = 4096
T = 256

TR = 32
CAP = T * TOPK + E * TR
FT = 1024
NF = F // FT


def _ffn_body(astart_ref, nt_ref, xs_ref, w1_ref, w3_ref, w2_ref, ys_ref):
    e = pl.program_id(0)
    f = pl.program_id(1)
    base = astart_ref[e]
    nt = nt_ref[e]
    def tile(i, carry):
        r0 = pl.multiple_of(base + i * TR, TR)
        xt = xs_ref[pl.ds(r0, TR), :]
        h = jnp.dot(xt, w1_ref[0], preferred_element_type=jnp.float32)
        g = jnp.dot(xt, w3_ref[0], preferred_element_type=jnp.float32)
        act = (h * jax.nn.sigmoid(h)) * g
        y = jnp.dot(act, w2_ref[0],
                    preferred_element_type=jnp.float32)

        @pl.when(f == 0)
        def _():
            ys_ref[pl.ds(r0, TR), :] = y

        @pl.when(f != 0)
        def _():
            ys_ref[pl.ds(r0, TR), :] = ys_ref[pl.ds(r0, TR), :] + y

        return carry

    jax.lax.fori_loop(0, nt, tile, 0)


def _expert_ffn(xs, w1, w3, w2, astart, ntiles):
    return pl.pallas_call(
        _ffn_body,
        grid=(E, NF),
        in_specs=[
            pl.BlockSpec(memory_space=pltpu.SMEM),
            pl.BlockSpec(memory_space=pltpu.SMEM),
            pl.BlockSpec((CAP, D), lambda e, f: (0, 0)),
            pl.BlockSpec((1, D, FT), lambda e, f: (e, 0, f)),
            pl.BlockSpec((1, D, FT), lambda e, f: (e, 0, f)),
            pl.BlockSpec((1, FT, D), lambda e, f: (e, f, 0)),
        ],
        out_specs=pl.BlockSpec((CAP, D), lambda e, f: (0, 0)),
        out_shape=jax.ShapeDtypeStruct((CAP, D), jnp.float32),
    )(astart, ntiles, xs, w1, w3, w2)


def kernel(hidden_states, gate_w, w1, w3, w2):
    b, s, d = hidden_states.shape
    x = hidden_states.reshape(-1, d)

    logits = x @ gate_w
    m0 = jnp.max(logits, axis=1)
    e0 = jnp.argmax(logits, axis=1).astype(jnp.int32)
    masked = jnp.where(jax.nn.one_hot(e0, E, dtype=jnp.bool_), -jnp.inf, logits)
    m1 = jnp.max(masked, axis=1)
    e1 = jnp.argmax(masked, axis=1).astype(jnp.int32)
    wn0 = 1.0 / (1.0 + jnp.exp(m1 - m0))
    wn1 = 1.0 - wn0

    eid = jnp.concatenate([e0, e1])
    onehot = jax.nn.one_hot(eid, E, dtype=jnp.int32)
    counts = onehot.sum(axis=0)
    aligned = ((counts + TR - 1) // TR) * TR
    astart = jnp.concatenate([jnp.zeros((1,), jnp.int32),
                              jnp.cumsum(aligned)[:-1].astype(jnp.int32)])
    ntiles = (aligned // TR).astype(jnp.int32)
    rank = (jnp.cumsum(onehot, axis=0) - onehot)
    rank = jnp.take_along_axis(rank, eid[:, None], axis=1)[:, 0]
    pos = astart[eid] + rank.astype(jnp.int32)
    tok = jnp.mod(jnp.arange(2 * T, dtype=jnp.int32), T)
    tok_ids = jnp.zeros((CAP,), jnp.int32).at[pos].set(tok)
    slot0, slot1 = pos[:T], pos[T:]

    xs = x[tok_ids]

    ys = _expert_ffn(xs, w1, w3, w2, astart, ntiles)

    out = wn0[:, None] * ys[slot0] + wn1[:, None] * ys[slot1]
    return out.reshape(b, s, d).astype(hidden_states.dtype), logits
